# baseline (speedup 1.0000x reference)
import functools

import jax
import jax.numpy as jnp
from jax import lax
from jax.experimental import pallas as pl
from jax.experimental.pallas import tpu as pltpu
from jax.experimental.pallas import tpu_sc as plsc

N_EDGES = 320000
EMB = 128
NLUT = 60


def _lut_body(w0_ref, w1_ref, w2_ref, lut_ref):
    r0 = lax.broadcasted_iota(jnp.int32, (NLUT, 5), 0)
    c0 = lax.broadcasted_iota(jnp.int32, (NLUT, 5), 1)
    oh0 = (r0 // 12 == c0).astype(jnp.float32)
    r1 = lax.broadcasted_iota(jnp.int32, (NLUT, 6), 0)
    c1 = lax.broadcasted_iota(jnp.int32, (NLUT, 6), 1)
    oh1 = ((r1 // 2) % 6 == c1).astype(jnp.float32)
    r2 = lax.broadcasted_iota(jnp.int32, (NLUT, 2), 0)
    c2 = lax.broadcasted_iota(jnp.int32, (NLUT, 2), 1)
    oh2 = (r2 % 2 == c2).astype(jnp.float32)
    lut_ref[...] = (
        jnp.dot(oh0, w0_ref[...], preferred_element_type=jnp.float32)
        + jnp.dot(oh1, w1_ref[...], preferred_element_type=jnp.float32)
        + jnp.dot(oh2, w2_ref[...], preferred_element_type=jnp.float32)
    )


_lut_call = pl.pallas_call(
    _lut_body,
    out_shape=jax.ShapeDtypeStruct((NLUT, EMB), jnp.float32),
)

NW = 32
ROWS_PER_W = N_EDGES // NW
SLAB = 200
NSLAB = ROWS_PER_W // SLAB
SLAB_W = SLAB * EMB


@functools.cache
def _get_sc_gather():
    @functools.partial(
        pl.kernel,
        out_type=jax.ShapeDtypeStruct((N_EDGES * EMB,), jnp.float32),
        mesh=plsc.VectorSubcoreMesh(core_axis_name="c", subcore_axis_name="s"),
        compiler_params=pltpu.CompilerParams(needs_layout_passes=False),
        scratch_types=[
            pltpu.VMEM_SHARED((16 * ROWS_PER_W,), jnp.int32),
            [pltpu.VMEM((ROWS_PER_W,), jnp.int32)] * 3,
            pltpu.VMEM((ROWS_PER_W,), jnp.int32),
            pltpu.VMEM((NLUT * EMB,), jnp.float32),
            pltpu.VMEM((SLAB_W,), jnp.float32),
            pltpu.VMEM((SLAB_W,), jnp.float32),
            [pltpu.SMEM((SLAB,), jnp.int32)] * 2,
            [pltpu.SemaphoreType.DMA] * 2,
            pltpu.SemaphoreType.DMA,
            pltpu.SemaphoreType.DMA,
        ],
    )
    def _sc_gather(
        a0_hbm, a1_hbm, a2_hbm, lut_hbm, out_hbm,
        cidx_sh, a_t, cidx_t, lut_v, buf0, buf1, sms, ssems, o0, o1,
    ):
        cid = lax.axis_index("c")
        sid = lax.axis_index("s")
        wid = sid * 2 + cid
        out_base = wid * ROWS_PER_W * EMB
        sh_base = sid * ROWS_PER_W
        for k, a_hbm in enumerate((a0_hbm, a1_hbm, a2_hbm)):
            pltpu.sync_copy(
                a_hbm.at[pl.ds(wid * ROWS_PER_W, ROWS_PER_W)], a_t[k]
            )

        def fold(j, carry):
            v0 = a_t[0][pl.ds(j * 16, 16)]
            v1 = a_t[1][pl.ds(j * 16, 16)]
            v2 = a_t[2][pl.ds(j * 16, 16)]
            cidx_t[pl.ds(j * 16, 16)] = (v0 * 12 + v1 * 2 + v2) * EMB
            return carry

        lax.fori_loop(0, ROWS_PER_W // 16, fold, 0)
        pltpu.sync_copy(cidx_t, cidx_sh.at[pl.ds(sh_base, ROWS_PER_W)])
        pltpu.sync_copy(lut_hbm, lut_v)

        def issue_sm(s, k):
            pltpu.async_copy(
                cidx_sh.at[pl.ds(sh_base + s * SLAB, SLAB)], sms[k], ssems[k]
            )

        def wait_sm(s, k):
            pltpu.make_async_copy(
                cidx_sh.at[pl.ds(sh_base + s * SLAB, SLAB)], sms[k], ssems[k]
            ).wait()

        def fill(s, buf, k):
            wait_sm(s, k)

            @pl.when(s + 1 < NSLAB)
            def _():
                issue_sm(s + 1, 1 - k)

            sm = sms[k]

            @plsc.parallel_loop(0, SLAB, step=1, unroll=8)
            def edge(e):
                base = sm[e]
                ebase = e * EMB
                for d in range(EMB // 16):
                    buf[pl.ds(ebase + d * 16, 16)] = lut_v[
                        pl.ds(base + d * 16, 16)
                    ]

        def start_out(s, buf, sem):
            pltpu.async_copy(
                buf, out_hbm.at[pl.ds(out_base + s * SLAB_W, SLAB_W)], sem
            )

        def wait_out(s, buf, sem):
            pltpu.make_async_copy(
                buf, out_hbm.at[pl.ds(out_base + s * SLAB_W, SLAB_W)], sem
            ).wait()

        issue_sm(0, 0)
        fill(0, buf0, 0)
        start_out(0, buf0, o0)
        fill(1, buf1, 1)
        start_out(1, buf1, o1)

        def pair(p, carry):
            s0 = 2 * p + 2
            wait_out(s0 - 2, buf0, o0)
            fill(s0, buf0, 0)
            start_out(s0, buf0, o0)
            s1 = s0 + 1
            wait_out(s1 - 2, buf1, o1)
            fill(s1, buf1, 1)
            start_out(s1, buf1, o1)
            return carry

        lax.fori_loop(0, (NSLAB - 2) // 2, pair, 0)
        wait_out(NSLAB - 2, buf0, o0)
        wait_out(NSLAB - 1, buf1, o1)

    return _sc_gather


def kernel(edge_attr, W0, W1, W2):
    lut = _lut_call(W0, W1, W2)
    a0 = edge_attr[:, 0]
    a1 = edge_attr[:, 1]
    a2 = edge_attr[:, 2]
    out = _get_sc_gather()(a0, a1, a2, lut.reshape(-1))
    return out.reshape(N_EDGES, EMB)


# speedup vs baseline: 1.0041x; 1.0041x over previous
import functools

import jax
import jax.numpy as jnp
from jax import lax
from jax.experimental import pallas as pl
from jax.experimental.pallas import tpu as pltpu
from jax.experimental.pallas import tpu_sc as plsc

N_EDGES = 320000
EMB = 128
NLUT = 60


def _lut_body(w0_ref, w1_ref, w2_ref, lut_ref):
    r0 = lax.broadcasted_iota(jnp.int32, (NLUT, 5), 0)
    c0 = lax.broadcasted_iota(jnp.int32, (NLUT, 5), 1)
    oh0 = (r0 // 12 == c0).astype(jnp.float32)
    r1 = lax.broadcasted_iota(jnp.int32, (NLUT, 6), 0)
    c1 = lax.broadcasted_iota(jnp.int32, (NLUT, 6), 1)
    oh1 = ((r1 // 2) % 6 == c1).astype(jnp.float32)
    r2 = lax.broadcasted_iota(jnp.int32, (NLUT, 2), 0)
    c2 = lax.broadcasted_iota(jnp.int32, (NLUT, 2), 1)
    oh2 = (r2 % 2 == c2).astype(jnp.float32)
    lut_ref[...] = (
        jnp.dot(oh0, w0_ref[...], preferred_element_type=jnp.float32)
        + jnp.dot(oh1, w1_ref[...], preferred_element_type=jnp.float32)
        + jnp.dot(oh2, w2_ref[...], preferred_element_type=jnp.float32)
    )


_lut_call = pl.pallas_call(
    _lut_body,
    out_shape=jax.ShapeDtypeStruct((NLUT, EMB), jnp.float32),
)

NW = 32
ROWS_PER_W = N_EDGES // NW
SLAB = 200
NSLAB = ROWS_PER_W // SLAB
SLAB_W = SLAB * EMB


@functools.cache
def _get_sc_gather():
    @functools.partial(
        pl.kernel,
        out_type=jax.ShapeDtypeStruct((N_EDGES * EMB,), jnp.float32),
        mesh=plsc.VectorSubcoreMesh(core_axis_name="c", subcore_axis_name="s"),
        compiler_params=pltpu.CompilerParams(needs_layout_passes=False),
        scratch_types=[
            pltpu.VMEM_SHARED((16 * ROWS_PER_W,), jnp.int32),
            [pltpu.VMEM((ROWS_PER_W,), jnp.int32)] * 3,
            pltpu.VMEM((ROWS_PER_W,), jnp.int32),
            pltpu.VMEM((NLUT * EMB,), jnp.float32),
            pltpu.VMEM((SLAB_W,), jnp.float32),
            pltpu.VMEM((SLAB_W,), jnp.float32),
            [pltpu.SMEM((SLAB,), jnp.int32)] * 2,
            [pltpu.SemaphoreType.DMA] * 2,
            pltpu.SemaphoreType.DMA,
            pltpu.SemaphoreType.DMA,
        ],
    )
    def _sc_gather(
        a0_hbm, a1_hbm, a2_hbm, lut_hbm, out_hbm,
        cidx_sh, a_t, cidx_t, lut_v, buf0, buf1, sms, ssems, o0, o1,
    ):
        cid = lax.axis_index("c")
        sid = lax.axis_index("s")
        wid = sid * 2 + cid
        out_base = wid * ROWS_PER_W * EMB
        sh_base = sid * ROWS_PER_W
        for k, a_hbm in enumerate((a0_hbm, a1_hbm, a2_hbm)):
            pltpu.sync_copy(
                a_hbm.at[pl.ds(wid * ROWS_PER_W, ROWS_PER_W)], a_t[k]
            )

        def fold(j, carry):
            v0 = a_t[0][pl.ds(j * 16, 16)]
            v1 = a_t[1][pl.ds(j * 16, 16)]
            v2 = a_t[2][pl.ds(j * 16, 16)]
            cidx_t[pl.ds(j * 16, 16)] = (v0 * 12 + v1 * 2 + v2) * EMB
            return carry

        lax.fori_loop(0, ROWS_PER_W // 16, fold, 0)
        pltpu.sync_copy(cidx_t, cidx_sh.at[pl.ds(sh_base, ROWS_PER_W)])
        pltpu.sync_copy(lut_hbm, lut_v)

        def issue_sm(s, k):
            pltpu.async_copy(
                cidx_sh.at[pl.ds(sh_base + s * SLAB, SLAB)], sms[k], ssems[k]
            )

        def wait_sm(s, k):
            pltpu.make_async_copy(
                cidx_sh.at[pl.ds(sh_base + s * SLAB, SLAB)], sms[k], ssems[k]
            ).wait()

        def fill(s, buf, k):
            wait_sm(s, k)

            @pl.when(s + 1 < NSLAB)
            def _():
                issue_sm(s + 1, 1 - k)

            sm = sms[k]

            @plsc.parallel_loop(0, SLAB, step=1, unroll=4)
            def edge(e):
                base = sm[e]
                ebase = e * EMB
                for d in range(EMB // 16):
                    buf[pl.ds(ebase + d * 16, 16)] = lut_v[
                        pl.ds(base + d * 16, 16)
                    ]

        def start_out(s, buf, sem):
            pltpu.async_copy(
                buf, out_hbm.at[pl.ds(out_base + s * SLAB_W, SLAB_W)], sem
            )

        def wait_out(s, buf, sem):
            pltpu.make_async_copy(
                buf, out_hbm.at[pl.ds(out_base + s * SLAB_W, SLAB_W)], sem
            ).wait()

        issue_sm(0, 0)
        fill(0, buf0, 0)
        start_out(0, buf0, o0)
        fill(1, buf1, 1)
        start_out(1, buf1, o1)

        def pair(p, carry):
            s0 = 2 * p + 2
            wait_out(s0 - 2, buf0, o0)
            fill(s0, buf0, 0)
            start_out(s0, buf0, o0)
            s1 = s0 + 1
            wait_out(s1 - 2, buf1, o1)
            fill(s1, buf1, 1)
            start_out(s1, buf1, o1)
            return carry

        lax.fori_loop(0, (NSLAB - 2) // 2, pair, 0)
        wait_out(NSLAB - 2, buf0, o0)
        wait_out(NSLAB - 1, buf1, o1)

    return _sc_gather


def kernel(edge_attr, W0, W1, W2):
    lut = _lut_call(W0, W1, W2)
    a0 = edge_attr[:, 0]
    a1 = edge_attr[:, 1]
    a2 = edge_attr[:, 2]
    out = _get_sc_gather()(a0, a1, a2, lut.reshape(-1))
    return out.reshape(N_EDGES, EMB)
